# Initial kernel scaffold; baseline (speedup 1.0000x reference)
import jax
import jax.numpy as jnp
from jax.experimental import pallas as pl


def kernel(feats_bcn, coords_bnc3):
    raise NotImplementedError("write your pallas kernel here")


# speedup vs baseline: 10.2879x; 10.2879x over previous
import dataclasses

import jax
import jax.numpy as jnp
from jax import lax
from jax.experimental import pallas as pl
from jax.experimental.pallas import tpu as pltpu
from jax.experimental.pallas import tpu_sc as plsc

B, C, N, R = 8, 64, 65536, 32
V = R * R * R
INVALID = V
CHUNK = 2048
NCHUNK = N // CHUNK
NBUF = 4
LANES = 16
NSUB = 16


def _idx_body(c_ref, idx_ref):
    def axis(i):
        c01 = (c_ref[i] + 1.0) * 0.5
        ok = (c01 >= 0.0) & (c01 <= 1.0)
        q = jnp.round(jnp.clip(c01, 0.0, 1.0) * float(R - 1)).astype(jnp.int32)
        return ok, q

    okx, xi = axis(0)
    oky, yi = axis(1)
    okz, zi = axis(2)
    valid = okx & oky & okz
    flat = xi * (R * R) + yi * R + zi
    idx_ref[...] = jnp.where(valid, flat, INVALID)


def _voxel_index(coords_t):
    return pl.pallas_call(
        _idx_body,
        out_shape=jax.ShapeDtypeStruct((B, N), jnp.int32),
    )(coords_t)


def _sc_body(feats_hbm, idx_hbm, sums_hbm, cnts_hbm,
             grid0, grid1, gridc, idxvs, valvs, semis, semvs):
    core = lax.axis_index("c")
    sub = lax.axis_index("s")
    c0 = core * (C // 2) + sub * 2

    @pl.loop(0, B)
    def _batch(b):
        def start(j, t):
            base = j * CHUNK
            pltpu.make_async_copy(
                idx_hbm.at[b, pl.ds(base, CHUNK)], idxvs[t], semis[t]).start()
            pltpu.make_async_copy(
                feats_hbm.at[b, pl.ds(c0, 2), pl.ds(base, CHUNK)],
                valvs[t], semvs[t]).start()

        def wait(t):
            pltpu.make_async_copy(
                idx_hbm.at[b, pl.ds(0, CHUNK)], idxvs[t], semis[t]).wait()
            pltpu.make_async_copy(
                feats_hbm.at[b, pl.ds(0, 2), pl.ds(0, CHUNK)],
                valvs[t], semvs[t]).wait()

        def compute(j, t):
            idxv, valv = idxvs[t], valvs[t]

            @plsc.parallel_loop(0, CHUNK, step=LANES, unroll=8)
            def _grp(k):
                iv = idxv[pl.ds(k, LANES)]
                m = iv < INVALID
                v0 = valv[0, pl.ds(k, LANES)]
                v1 = valv[1, pl.ds(k, LANES)]
                plsc.addupdate_scatter(grid0, [iv], v0, mask=m)
                plsc.addupdate_scatter(grid1, [iv], v1, mask=m)

            @pl.when((sub < 8) & (j % NSUB == core * 8 + sub))
            def _cnt():
                ones = jnp.ones((LANES,), jnp.float32)

                @plsc.parallel_loop(0, CHUNK, step=LANES, unroll=8)
                def _grp2(k):
                    iv = idxv[pl.ds(k, LANES)]
                    m = iv < INVALID
                    plsc.addupdate_scatter(gridc, [iv], ones, mask=m)

        for t in range(NBUF):
            start(t, t)

        zz = jnp.zeros((LANES,), jnp.float32)

        @plsc.parallel_loop(0, V, step=LANES, unroll=8)
        def _zero(i):
            grid0[pl.ds(i, LANES)] = zz
            grid1[pl.ds(i, LANES)] = zz

        @pl.when(sub < 8)
        def _zeroc():
            @plsc.parallel_loop(0, V, step=LANES, unroll=8)
            def _zero2(i):
                gridc[pl.ds(i, LANES)] = zz

        @pl.loop(0, NCHUNK, step=NBUF)
        def _chunk(j):
            for t in range(NBUF):
                wait(t)
                compute(j + t, t)

                @pl.when(j + t + NBUF < NCHUNK)
                def _pre():
                    start(j + t + NBUF, t)

        pltpu.sync_copy(grid0.at[pl.ds(0, V)], sums_hbm.at[b, c0])
        pltpu.sync_copy(grid1.at[pl.ds(0, V)], sums_hbm.at[b, c0 + 1])

        @pl.when(sub < 8)
        def _copyc():
            pltpu.sync_copy(gridc.at[pl.ds(0, V)],
                            cnts_hbm.at[b, core * 8 + sub])


def _sc_scatter(feats_bcn, idx):
    mesh = plsc.VectorSubcoreMesh(core_axis_name="c", subcore_axis_name="s")
    cp = dataclasses.replace(pltpu.CompilerParams(), needs_layout_passes=False)
    f = pl.kernel(
        _sc_body,
        out_type=(jax.ShapeDtypeStruct((B, C, V), jnp.float32),
                  jax.ShapeDtypeStruct((B, NSUB, V), jnp.float32)),
        mesh=mesh,
        scratch_types=[
            pltpu.VMEM((V + LANES,), jnp.float32),
            pltpu.VMEM((V + LANES,), jnp.float32),
            pltpu.VMEM((V + LANES,), jnp.float32),
            tuple(pltpu.VMEM((CHUNK,), jnp.int32) for _ in range(NBUF)),
            tuple(pltpu.VMEM((2, CHUNK), jnp.float32) for _ in range(NBUF)),
            tuple(pltpu.SemaphoreType.DMA for _ in range(NBUF)),
            tuple(pltpu.SemaphoreType.DMA for _ in range(NBUF)),
        ],
        compiler_params=cp,
    )
    return f(feats_bcn, idx)


def _norm_body(sums_ref, cnts_ref, out_ref):
    cnt = jnp.sum(cnts_ref[0], axis=0, keepdims=True)
    recip = 1.0 / jnp.maximum(cnt, 1.0)
    scaled = sums_ref[0] * recip
    out_ref[0] = scaled.T


def _normalize(sums, cnts):
    return pl.pallas_call(
        _norm_body,
        grid=(B,),
        in_specs=[
            pl.BlockSpec((1, C, V), lambda b: (b, 0, 0)),
            pl.BlockSpec((1, NSUB, V), lambda b: (b, 0, 0)),
        ],
        out_specs=pl.BlockSpec((1, V, C), lambda b: (b, 0, 0)),
        out_shape=jax.ShapeDtypeStruct((B, V, C), jnp.float32),
    )(sums, cnts)


def kernel(feats_bcn, coords_bnc3):
    coords_t = jnp.transpose(coords_bnc3, (2, 0, 1))
    idx = _voxel_index(coords_t)
    sums, cnts = _sc_scatter(feats_bcn, idx)
    vox_t = _normalize(sums, cnts)
    return jnp.transpose(vox_t, (0, 2, 1)).reshape(B, C, R, R, R)
